# baseline (speedup 1.0000x reference)
import functools

import jax
import jax.numpy as jnp
from jax import lax
from jax.experimental import pallas as pl
from jax.experimental.pallas import tpu as pltpu
from jax.experimental.pallas import tpu_sc as plsc

N = 16384
H = 128
K = int(N * 0.1)

NC = 2
NS = 16
NW = NC * NS
HPW = H // NW
NVEC = N // 16


def _tc_transform_body(p_ref, t_ref, o_ref):
    p = p_ref[...]
    t = t_ref[...]
    bits = lax.bitcast_convert_type(p, jnp.uint32)
    key = bits ^ ((bits >> jnp.uint32(31)) | jnp.uint32(0x80000000))
    lane = lax.broadcasted_iota(jnp.uint32, p.shape, 0) & jnp.uint32(15)
    w = ((key >> jnp.uint32(24)) << jnp.uint32(20)) \
        | (lane << jnp.uint32(28)) | (lane << jnp.uint32(16)) \
        | ((key >> jnp.uint32(16)) & jnp.uint32(0xFF)) << jnp.uint32(8) \
        | (t > 0).astype(jnp.uint32)
    o_ref[...] = w.T


def _tc_transform(preds, targets):
    blk = 2048
    return pl.pallas_call(
        _tc_transform_body,
        grid=(N // blk,),
        in_specs=[
            pl.BlockSpec((blk, H), lambda i: (i, 0)),
            pl.BlockSpec((blk, H), lambda i: (i, 0)),
        ],
        out_specs=pl.BlockSpec((H, blk), lambda i: (0, i)),
        out_shape=jax.ShapeDtypeStruct((H, N), jnp.uint32),
    )(preds, targets)


def _suffix_and_search(hist_v, s_v, lane, rank):
    zero16 = jnp.zeros((16,), jnp.int32)

    carry = jnp.int32(0)
    num_ge = zero16
    flats = [None] * 16
    for j in range(15, -1, -1):
        acc = zero16
        for p in range(16):
            off = p * 256 + j * 16
            acc = acc + hist_v[pl.ds(off, 16)]
            hist_v[pl.ds(off, 16)] = zero16
        flats[j] = acc
        suf = lax.rev(plsc.cumsum(lax.rev(acc, (0,))), (0,)) + carry
        s_v[pl.ds(j * 16, 16)] = suf
        carry = carry + jnp.sum(acc)
        num_ge = num_ge + ((suf >> 16) >= rank).astype(jnp.int32)

    p = jnp.sum(num_ge) - 1
    pos = p & 15
    q = p - pos
    v0 = s_v[pl.ds(q, 16)]
    msk = lane == pos
    t_in = jnp.max(jnp.where(msk, v0, 0))
    fsel = flats[0]
    for j in range(1, 16):
        fsel = lax.select((p >> 4) == j, flats[j], fsel)
    f_p = jnp.max(jnp.where(msk, fsel, 0))
    above = (t_in >> 16) - (f_p >> 16)
    hits_hi = (t_in & 0xFFFF) - (f_p & 0xFFFF)
    e_cnt = f_p >> 16
    e_pos = f_p & 0xFFFF
    return p, above, hits_hi, e_cnt, e_pos


def _sc_body(keys_hbm, out_hbm, keys_v, hist_v, s_v, outv_v,
             sem0, sem1, sem2, sem3):
    wid = lax.axis_index("s") * NC + lax.axis_index("c")
    pltpu.sync_copy(keys_hbm.at[pl.ds(wid * HPW, HPW)], keys_v)

    lane = lax.broadcasted_iota(jnp.int32, (16,), 0)
    zero16 = jnp.zeros((16,), jnp.int32)

    @plsc.parallel_loop(0, 256 * 16, 16, unroll=8)
    def _(off):
        hist_v[pl.ds(off, 16)] = zero16

    res_vec = jnp.zeros((16,), jnp.float32)
    for h in range(HPW):
        @plsc.parallel_loop(0, N, 16, unroll=8)
        def _(off):
            w = keys_v[h, pl.ds(off, 16)]
            idx = plsc.bitcast(w >> jnp.uint32(20), jnp.int32)
            val = plsc.bitcast((w & jnp.uint32(1)) | jnp.uint32(0x10000),
                               jnp.int32)
            plsc.addupdate_scatter(hist_v, [idx], val)
        p1b, above1, hits1, _, _ = _suffix_and_search(hist_v, s_v, lane, K)
        rank1 = K - above1

        p1vec = plsc.bitcast(lane * 256 + p1b, jnp.uint32)

        @plsc.parallel_loop(0, N, 16, unroll=8)
        def _(off):
            w = keys_v[h, pl.ds(off, 16)]
            idx = plsc.bitcast((w >> jnp.uint32(8)) & jnp.uint32(0xFFF),
                               jnp.int32)
            val = plsc.bitcast((w & jnp.uint32(1)) | jnp.uint32(0x10000),
                               jnp.int32)
            plsc.addupdate_scatter(hist_v, [idx], val,
                                   mask=(w >> jnp.uint32(20)) == p1vec)
        _, above2, hits2, e_cnt, e_pos = _suffix_and_search(
            hist_v, s_v, lane, rank1)
        rank2 = rank1 - above2

        num_v = jnp.broadcast_to(
            ((hits1 + hits2) * e_cnt + rank2 * e_pos).astype(jnp.float32),
            (16,))
        den_v = jnp.broadcast_to((e_cnt * K).astype(jnp.float32), (16,))
        res_vec = jnp.where(lane == h, num_v / den_v, res_vec)

    outv_v[...] = res_vec
    pltpu.sync_copy(outv_v, out_hbm.at[wid])


@functools.partial(jax.jit)
def _sc_topk_hitrate(keys):
    mesh = plsc.VectorSubcoreMesh(core_axis_name="c", subcore_axis_name="s",
                                  num_cores=NC, num_subcores=NS)
    return pl.kernel(
        _sc_body,
        out_type=jax.ShapeDtypeStruct((NW, 16), jnp.float32),
        mesh=mesh,
        compiler_params=pltpu.CompilerParams(needs_layout_passes=False),
        scratch_types=[
            pltpu.VMEM((HPW, N), jnp.uint32),
            pltpu.VMEM((256 * 16,), jnp.int32),
            pltpu.VMEM((256,), jnp.int32),
            pltpu.VMEM((16,), jnp.float32),
            pltpu.SemaphoreType.DMA,
            pltpu.SemaphoreType.DMA,
            pltpu.SemaphoreType.DMA,
            pltpu.SemaphoreType.DMA,
        ],
    )(keys)


def kernel(preds, targets):
    keys = _tc_transform(preds, targets)
    out = _sc_topk_hitrate(keys)
    return out[:, :HPW].reshape(H)


# speedup vs baseline: 1.2414x; 1.2414x over previous
import functools

import jax
import jax.numpy as jnp
from jax import lax
from jax.experimental import pallas as pl
from jax.experimental.pallas import tpu as pltpu
from jax.experimental.pallas import tpu_sc as plsc

N = 16384
H = 128
K = int(N * 0.1)

NC = 2
NS = 16
NW = NC * NS
HPW = H // NW
NVEC = N // 16


def _tc_transform_body(p_ref, t_ref, o_ref):
    p = p_ref[...]
    t = t_ref[...]
    bits = lax.bitcast_convert_type(p, jnp.uint32)
    key = bits ^ ((bits >> jnp.uint32(31)) | jnp.uint32(0x80000000))
    lane = lax.broadcasted_iota(jnp.uint32, p.shape, 0) & jnp.uint32(15)
    w = ((key >> jnp.uint32(24)) << jnp.uint32(24)) \
        | (lane << jnp.uint32(20)) | (lane << jnp.uint32(8)) \
        | ((key >> jnp.uint32(16)) & jnp.uint32(0xFF)) << jnp.uint32(12) \
        | (t > 0).astype(jnp.uint32)
    o_ref[...] = w.T


def _tc_transform(preds, targets):
    blk = 2048
    return pl.pallas_call(
        _tc_transform_body,
        grid=(N // blk,),
        in_specs=[
            pl.BlockSpec((blk, H), lambda i: (i, 0)),
            pl.BlockSpec((blk, H), lambda i: (i, 0)),
        ],
        out_specs=pl.BlockSpec((H, blk), lambda i: (0, i)),
        out_shape=jax.ShapeDtypeStruct((H, N), jnp.uint32),
    )(preds, targets)


def _suffix_and_search(hist_v, s_v, lane, rank):
    zero16 = jnp.zeros((16,), jnp.int32)

    def sfx(i, acc):
        br = 255 - i
        acc = acc + hist_v[pl.ds(br * 16, 16)]
        s_v[pl.ds(br * 16, 16)] = acc
        hist_v[pl.ds(br * 16, 16)] = zero16
        return acc

    lax.fori_loop(0, 256, sfx, zero16, unroll=8)

    def bs(_, lohi):
        lo, hi = lohi
        mid = (lo + hi + 1) // 2
        c = jnp.sum(s_v[pl.ds(mid * 16, 16)]) >> 16
        take = c >= rank
        return (lax.select(take, mid, lo), lax.select(take, hi, mid - 1))

    p, _ = lax.fori_loop(0, 8, bs, (jnp.int32(0), jnp.int32(255)))
    t_in = jnp.sum(s_v[pl.ds(p * 16, 16)])
    t_ab = jnp.sum(s_v[pl.ds(p * 16 + 16, 16)])
    above = t_ab >> 16
    hits_hi = t_ab & 0xFFFF
    e_cnt = (t_in >> 16) - above
    e_pos = (t_in & 0xFFFF) - hits_hi
    return p, above, hits_hi, e_cnt, e_pos


def _sc_body(keys_hbm, out_hbm, keys_v, hist_v, s_v, outv_v):
    wid = lax.axis_index("s") * NC + lax.axis_index("c")
    pltpu.sync_copy(keys_hbm.at[pl.ds(wid * HPW, HPW)], keys_v)

    lane = lax.broadcasted_iota(jnp.int32, (16,), 0)
    zero16 = jnp.zeros((16,), jnp.int32)

    @plsc.parallel_loop(0, 256 * 16, 16, unroll=8)
    def _(off):
        hist_v[pl.ds(off, 16)] = zero16

    s_v[pl.ds(256 * 16, 16)] = zero16

    res_vec = jnp.zeros((16,), jnp.float32)
    for h in range(HPW):
        @plsc.parallel_loop(0, N, 16, unroll=8)
        def _(off):
            w = keys_v[h, pl.ds(off, 16)]
            idx = plsc.bitcast(w >> jnp.uint32(20), jnp.int32)
            val = plsc.bitcast((w & jnp.uint32(1)) | jnp.uint32(0x10000),
                               jnp.int32)
            plsc.addupdate_scatter(hist_v, [idx], val)
        p1b, above1, hits1, _, _ = _suffix_and_search(hist_v, s_v, lane, K)
        rank1 = K - above1

        p1vec = plsc.bitcast(lane + p1b * 16, jnp.uint32)

        @plsc.parallel_loop(0, N, 16, unroll=8)
        def _(off):
            w = keys_v[h, pl.ds(off, 16)]
            idx = plsc.bitcast((w >> jnp.uint32(8)) & jnp.uint32(0xFFF),
                               jnp.int32)
            val = plsc.bitcast((w & jnp.uint32(1)) | jnp.uint32(0x10000),
                               jnp.int32)
            plsc.addupdate_scatter(hist_v, [idx], val,
                                   mask=(w >> jnp.uint32(20)) == p1vec)
        _, above2, hits2, e_cnt, e_pos = _suffix_and_search(
            hist_v, s_v, lane, rank1)
        rank2 = rank1 - above2

        num_v = jnp.broadcast_to(
            ((hits1 + hits2) * e_cnt + rank2 * e_pos).astype(jnp.float32),
            (16,))
        den_v = jnp.broadcast_to((e_cnt * K).astype(jnp.float32), (16,))
        res_vec = jnp.where(lane == h, num_v / den_v, res_vec)

    outv_v[...] = res_vec
    pltpu.sync_copy(outv_v, out_hbm.at[wid])


@functools.partial(jax.jit)
def _sc_topk_hitrate(keys):
    mesh = plsc.VectorSubcoreMesh(core_axis_name="c", subcore_axis_name="s",
                                  num_cores=NC, num_subcores=NS)
    return pl.kernel(
        _sc_body,
        out_type=jax.ShapeDtypeStruct((NW, 16), jnp.float32),
        mesh=mesh,
        compiler_params=pltpu.CompilerParams(needs_layout_passes=False),
        scratch_types=[
            pltpu.VMEM((HPW, N), jnp.uint32),
            pltpu.VMEM((256 * 16,), jnp.int32),
            pltpu.VMEM((257 * 16,), jnp.int32),
            pltpu.VMEM((16,), jnp.float32),
        ],
    )(keys)


def kernel(preds, targets):
    keys = _tc_transform(preds, targets)
    out = _sc_topk_hitrate(keys)
    return out[:, :HPW].reshape(H)


# speedup vs baseline: 1.2788x; 1.0301x over previous
import functools

import jax
import jax.numpy as jnp
from jax import lax
from jax.experimental import pallas as pl
from jax.experimental.pallas import tpu as pltpu
from jax.experimental.pallas import tpu_sc as plsc

N = 16384
H = 128
K = int(N * 0.1)

NC = 2
NS = 16
NW = NC * NS
HPW = H // NW
NVEC = N // 16


def _tc_transform_body(p_ref, t_ref, o_ref):
    p = p_ref[...]
    t = t_ref[...]
    bits = lax.bitcast_convert_type(p, jnp.uint32)
    key = bits ^ ((bits >> jnp.uint32(31)) | jnp.uint32(0x80000000))
    lane = lax.broadcasted_iota(jnp.uint32, p.shape, 0) & jnp.uint32(15)
    w = ((key >> jnp.uint32(24)) << jnp.uint32(24)) \
        | (lane << jnp.uint32(20)) | (lane << jnp.uint32(8)) \
        | ((key >> jnp.uint32(16)) & jnp.uint32(0xFF)) << jnp.uint32(12) \
        | (t > 0).astype(jnp.uint32)
    o_ref[...] = w.T


def _tc_transform(preds, targets):
    blk = 4096
    return pl.pallas_call(
        _tc_transform_body,
        grid=(N // blk,),
        in_specs=[
            pl.BlockSpec((blk, H), lambda i: (i, 0)),
            pl.BlockSpec((blk, H), lambda i: (i, 0)),
        ],
        out_specs=pl.BlockSpec((H, blk), lambda i: (0, i)),
        out_shape=jax.ShapeDtypeStruct((H, N), jnp.uint32),
    )(preds, targets)


def _suffix_and_search(hist_v, s_v, lane, rank):
    zero16 = jnp.zeros((16,), jnp.int32)

    def sfx(i, acc):
        br = 255 - i
        acc = acc + hist_v[pl.ds(br * 16, 16)]
        s_v[pl.ds(br * 16, 16)] = acc
        hist_v[pl.ds(br * 16, 16)] = zero16
        return acc

    lax.fori_loop(0, 256, sfx, zero16, unroll=8)

    def bs(_, lohi):
        lo, hi = lohi
        mid = (lo + hi + 1) // 2
        c = jnp.sum(s_v[pl.ds(mid * 16, 16)]) >> 16
        take = c >= rank
        return (lax.select(take, mid, lo), lax.select(take, hi, mid - 1))

    p, _ = lax.fori_loop(0, 8, bs, (jnp.int32(0), jnp.int32(255)))
    t_in = jnp.sum(s_v[pl.ds(p * 16, 16)])
    t_ab = jnp.sum(s_v[pl.ds(p * 16 + 16, 16)])
    above = t_ab >> 16
    hits_hi = t_ab & 0xFFFF
    e_cnt = (t_in >> 16) - above
    e_pos = (t_in & 0xFFFF) - hits_hi
    return p, above, hits_hi, e_cnt, e_pos


def _sc_body(keys_hbm, out_hbm, keys_v, hist_v, s_v, outv_v):
    wid = lax.axis_index("s") * NC + lax.axis_index("c")
    pltpu.sync_copy(keys_hbm.at[pl.ds(wid * HPW, HPW)], keys_v)

    lane = lax.broadcasted_iota(jnp.int32, (16,), 0)
    zero16 = jnp.zeros((16,), jnp.int32)

    @plsc.parallel_loop(0, 256 * 16, 16, unroll=8)
    def _(off):
        hist_v[pl.ds(off, 16)] = zero16

    s_v[pl.ds(256 * 16, 16)] = zero16

    res_vec = jnp.zeros((16,), jnp.float32)
    for h in range(HPW):
        @plsc.parallel_loop(0, N, 16, unroll=8)
        def _(off):
            w = keys_v[h, pl.ds(off, 16)]
            idx = plsc.bitcast(w >> jnp.uint32(20), jnp.int32)
            val = plsc.bitcast((w & jnp.uint32(1)) | jnp.uint32(0x10000),
                               jnp.int32)
            plsc.addupdate_scatter(hist_v, [idx], val)
        p1b, above1, hits1, _, _ = _suffix_and_search(hist_v, s_v, lane, K)
        rank1 = K - above1

        p1vec = plsc.bitcast(lane + p1b * 16, jnp.uint32)

        @plsc.parallel_loop(0, N, 16, unroll=8)
        def _(off):
            w = keys_v[h, pl.ds(off, 16)]
            idx = plsc.bitcast((w >> jnp.uint32(8)) & jnp.uint32(0xFFF),
                               jnp.int32)
            val = plsc.bitcast((w & jnp.uint32(1)) | jnp.uint32(0x10000),
                               jnp.int32)
            plsc.addupdate_scatter(hist_v, [idx], val,
                                   mask=(w >> jnp.uint32(20)) == p1vec)
        _, above2, hits2, e_cnt, e_pos = _suffix_and_search(
            hist_v, s_v, lane, rank1)
        rank2 = rank1 - above2

        num_v = jnp.broadcast_to(
            ((hits1 + hits2) * e_cnt + rank2 * e_pos).astype(jnp.float32),
            (16,))
        den_v = jnp.broadcast_to((e_cnt * K).astype(jnp.float32), (16,))
        res_vec = jnp.where(lane == h, num_v / den_v, res_vec)

    outv_v[...] = res_vec
    pltpu.sync_copy(outv_v, out_hbm.at[wid])


@functools.partial(jax.jit)
def _sc_topk_hitrate(keys):
    mesh = plsc.VectorSubcoreMesh(core_axis_name="c", subcore_axis_name="s",
                                  num_cores=NC, num_subcores=NS)
    return pl.kernel(
        _sc_body,
        out_type=jax.ShapeDtypeStruct((NW, 16), jnp.float32),
        mesh=mesh,
        compiler_params=pltpu.CompilerParams(needs_layout_passes=False),
        scratch_types=[
            pltpu.VMEM((HPW, N), jnp.uint32),
            pltpu.VMEM((256 * 16,), jnp.int32),
            pltpu.VMEM((257 * 16,), jnp.int32),
            pltpu.VMEM((16,), jnp.float32),
        ],
    )(keys)


def kernel(preds, targets):
    keys = _tc_transform(preds, targets)
    out = _sc_topk_hitrate(keys)
    return out[:, :HPW].reshape(H)


# speedup vs baseline: 1.2914x; 1.0099x over previous
import functools

import jax
import jax.numpy as jnp
from jax import lax
from jax.experimental import pallas as pl
from jax.experimental.pallas import tpu as pltpu
from jax.experimental.pallas import tpu_sc as plsc

N = 16384
H = 128
K = int(N * 0.1)

NC = 2
NS = 16
NW = NC * NS
HPW = H // NW
NVEC = N // 16


def _tc_transform_body(p_ref, t_ref, o_ref):
    p = p_ref[...]
    t = t_ref[...]
    bits = lax.bitcast_convert_type(p, jnp.uint32)
    key = bits ^ ((bits >> jnp.uint32(31)) | jnp.uint32(0x80000000))
    lane = lax.broadcasted_iota(jnp.uint32, p.shape, 0) & jnp.uint32(15)
    w = ((key >> jnp.uint32(24)) << jnp.uint32(24)) \
        | (lane << jnp.uint32(20)) | (lane << jnp.uint32(8)) \
        | ((key >> jnp.uint32(16)) & jnp.uint32(0xFF)) << jnp.uint32(12) \
        | (t > 0).astype(jnp.uint32)
    o_ref[...] = w.T


def _tc_transform(preds, targets):
    blk = 8192
    return pl.pallas_call(
        _tc_transform_body,
        grid=(N // blk,),
        in_specs=[
            pl.BlockSpec((blk, H), lambda i: (i, 0)),
            pl.BlockSpec((blk, H), lambda i: (i, 0)),
        ],
        out_specs=pl.BlockSpec((H, blk), lambda i: (0, i)),
        out_shape=jax.ShapeDtypeStruct((H, N), jnp.uint32),
    )(preds, targets)


def _suffix_and_search(hist_v, s_v, lane, rank):
    zero16 = jnp.zeros((16,), jnp.int32)

    def sfx(i, acc):
        br = 255 - i
        acc = acc + hist_v[pl.ds(br * 16, 16)]
        s_v[pl.ds(br * 16, 16)] = acc
        hist_v[pl.ds(br * 16, 16)] = zero16
        return acc

    lax.fori_loop(0, 256, sfx, zero16, unroll=8)

    def bs(_, lohi):
        lo, hi = lohi
        mid = (lo + hi + 1) // 2
        c = jnp.sum(s_v[pl.ds(mid * 16, 16)]) >> 16
        take = c >= rank
        return (lax.select(take, mid, lo), lax.select(take, hi, mid - 1))

    p, _ = lax.fori_loop(0, 8, bs, (jnp.int32(0), jnp.int32(255)))
    t_in = jnp.sum(s_v[pl.ds(p * 16, 16)])
    t_ab = jnp.sum(s_v[pl.ds(p * 16 + 16, 16)])
    above = t_ab >> 16
    hits_hi = t_ab & 0xFFFF
    e_cnt = (t_in >> 16) - above
    e_pos = (t_in & 0xFFFF) - hits_hi
    return p, above, hits_hi, e_cnt, e_pos


def _sc_body(keys_hbm, out_hbm, keys_v, hist_v, s_v, outv_v):
    wid = lax.axis_index("s") * NC + lax.axis_index("c")
    pltpu.sync_copy(keys_hbm.at[pl.ds(wid * HPW, HPW)], keys_v)

    lane = lax.broadcasted_iota(jnp.int32, (16,), 0)
    zero16 = jnp.zeros((16,), jnp.int32)

    @plsc.parallel_loop(0, 256 * 16, 16, unroll=8)
    def _(off):
        hist_v[pl.ds(off, 16)] = zero16

    s_v[pl.ds(256 * 16, 16)] = zero16

    res_vec = jnp.zeros((16,), jnp.float32)
    for h in range(HPW):
        @plsc.parallel_loop(0, N, 16, unroll=8)
        def _(off):
            w = keys_v[h, pl.ds(off, 16)]
            idx = plsc.bitcast(w >> jnp.uint32(20), jnp.int32)
            val = plsc.bitcast((w & jnp.uint32(1)) | jnp.uint32(0x10000),
                               jnp.int32)
            plsc.addupdate_scatter(hist_v, [idx], val)
        p1b, above1, hits1, _, _ = _suffix_and_search(hist_v, s_v, lane, K)
        rank1 = K - above1

        p1vec = plsc.bitcast(lane + p1b * 16, jnp.uint32)

        @plsc.parallel_loop(0, N, 16, unroll=8)
        def _(off):
            w = keys_v[h, pl.ds(off, 16)]
            idx = plsc.bitcast((w >> jnp.uint32(8)) & jnp.uint32(0xFFF),
                               jnp.int32)
            val = plsc.bitcast((w & jnp.uint32(1)) | jnp.uint32(0x10000),
                               jnp.int32)
            plsc.addupdate_scatter(hist_v, [idx], val,
                                   mask=(w >> jnp.uint32(20)) == p1vec)
        _, above2, hits2, e_cnt, e_pos = _suffix_and_search(
            hist_v, s_v, lane, rank1)
        rank2 = rank1 - above2

        num_v = jnp.broadcast_to(
            ((hits1 + hits2) * e_cnt + rank2 * e_pos).astype(jnp.float32),
            (16,))
        den_v = jnp.broadcast_to((e_cnt * K).astype(jnp.float32), (16,))
        res_vec = jnp.where(lane == h, num_v / den_v, res_vec)

    outv_v[...] = res_vec
    pltpu.sync_copy(outv_v, out_hbm.at[wid])


@functools.partial(jax.jit)
def _sc_topk_hitrate(keys):
    mesh = plsc.VectorSubcoreMesh(core_axis_name="c", subcore_axis_name="s",
                                  num_cores=NC, num_subcores=NS)
    return pl.kernel(
        _sc_body,
        out_type=jax.ShapeDtypeStruct((NW, 16), jnp.float32),
        mesh=mesh,
        compiler_params=pltpu.CompilerParams(needs_layout_passes=False),
        scratch_types=[
            pltpu.VMEM((HPW, N), jnp.uint32),
            pltpu.VMEM((256 * 16,), jnp.int32),
            pltpu.VMEM((257 * 16,), jnp.int32),
            pltpu.VMEM((16,), jnp.float32),
        ],
    )(keys)


def kernel(preds, targets):
    keys = _tc_transform(preds, targets)
    out = _sc_topk_hitrate(keys)
    return out[:, :HPW].reshape(H)


# speedup vs baseline: 1.3141x; 1.0175x over previous
import functools

import jax
import jax.numpy as jnp
from jax import lax
from jax.experimental import pallas as pl
from jax.experimental.pallas import tpu as pltpu
from jax.experimental.pallas import tpu_sc as plsc

N = 16384
H = 128
K = int(N * 0.1)

NC = 2
NS = 16
NW = NC * NS
HPW = H // NW
NVEC = N // 16


def _tc_transform_body(p_ref, t_ref, o_ref):
    p = p_ref[...]
    t = t_ref[...]
    bits = lax.bitcast_convert_type(p, jnp.uint32)
    key = bits ^ ((bits >> jnp.uint32(31)) | jnp.uint32(0x80000000))
    lane = lax.broadcasted_iota(jnp.uint32, p.shape, 0) & jnp.uint32(15)
    w = ((key >> jnp.uint32(24)) << jnp.uint32(24)) \
        | (lane << jnp.uint32(20)) | (lane << jnp.uint32(8)) \
        | ((key >> jnp.uint32(16)) & jnp.uint32(0xFF)) << jnp.uint32(12) \
        | (t > 0).astype(jnp.uint32)
    o_ref[...] = w.T


def _tc_transform(preds, targets):
    blk = 8192
    return pl.pallas_call(
        _tc_transform_body,
        grid=(N // blk,),
        in_specs=[
            pl.BlockSpec((blk, H), lambda i: (i, 0)),
            pl.BlockSpec((blk, H), lambda i: (i, 0)),
        ],
        out_specs=pl.BlockSpec((H, blk), lambda i: (0, i)),
        out_shape=jax.ShapeDtypeStruct((H, N), jnp.uint32),
    )(preds, targets)


def _suffix_and_search(hist_v, s_v, lane, rank):
    zero16 = jnp.zeros((16,), jnp.int32)

    def sfx(i, acc):
        br = 255 - i
        acc = acc + hist_v[pl.ds(br * 16, 16)]
        s_v[pl.ds(br * 16, 16)] = acc
        hist_v[pl.ds(br * 16, 16)] = zero16
        return acc

    lax.fori_loop(0, 256, sfx, zero16, unroll=8)

    def bs(_, lohi):
        lo, hi = lohi
        mid = (lo + hi + 1) // 2
        c = jnp.sum(s_v[pl.ds(mid * 16, 16)]) >> 16
        take = c >= rank
        return (lax.select(take, mid, lo), lax.select(take, hi, mid - 1))

    p, _ = lax.fori_loop(0, 8, bs, (jnp.int32(0), jnp.int32(255)))
    t_in = jnp.sum(s_v[pl.ds(p * 16, 16)])
    t_ab = jnp.sum(s_v[pl.ds(p * 16 + 16, 16)])
    above = t_ab >> 16
    hits_hi = t_ab & 0xFFFF
    e_cnt = (t_in >> 16) - above
    e_pos = (t_in & 0xFFFF) - hits_hi
    return p, above, hits_hi, e_cnt, e_pos


def _sc_body(keys_hbm, out_hbm, keys_v, hist_v, s_v, outv_v,
             sem0, sem1, sem2, sem3):
    wid = lax.axis_index("s") * NC + lax.axis_index("c")
    sems = [sem0, sem1, sem2, sem3]
    copies = [
        pltpu.async_copy(keys_hbm.at[wid * HPW + h], keys_v.at[h], sems[h])
        for h in range(HPW)
    ]

    lane = lax.broadcasted_iota(jnp.int32, (16,), 0)
    zero16 = jnp.zeros((16,), jnp.int32)

    @plsc.parallel_loop(0, 256 * 16, 16, unroll=8)
    def _(off):
        hist_v[pl.ds(off, 16)] = zero16

    s_v[pl.ds(256 * 16, 16)] = zero16

    res_vec = jnp.zeros((16,), jnp.float32)
    for h in range(HPW):
        copies[h].wait()

        @plsc.parallel_loop(0, N, 16, unroll=8)
        def _(off):
            w = keys_v[h, pl.ds(off, 16)]
            idx = plsc.bitcast(w >> jnp.uint32(20), jnp.int32)
            val = plsc.bitcast((w & jnp.uint32(1)) | jnp.uint32(0x10000),
                               jnp.int32)
            plsc.addupdate_scatter(hist_v, [idx], val)
        p1b, above1, hits1, _, _ = _suffix_and_search(hist_v, s_v, lane, K)
        rank1 = K - above1

        p1vec = plsc.bitcast(lane + p1b * 16, jnp.uint32)

        @plsc.parallel_loop(0, N, 16, unroll=8)
        def _(off):
            w = keys_v[h, pl.ds(off, 16)]
            idx = plsc.bitcast((w >> jnp.uint32(8)) & jnp.uint32(0xFFF),
                               jnp.int32)
            val = plsc.bitcast((w & jnp.uint32(1)) | jnp.uint32(0x10000),
                               jnp.int32)
            plsc.addupdate_scatter(hist_v, [idx], val,
                                   mask=(w >> jnp.uint32(20)) == p1vec)
        _, above2, hits2, e_cnt, e_pos = _suffix_and_search(
            hist_v, s_v, lane, rank1)
        rank2 = rank1 - above2

        num_v = jnp.broadcast_to(
            ((hits1 + hits2) * e_cnt + rank2 * e_pos).astype(jnp.float32),
            (16,))
        den_v = jnp.broadcast_to((e_cnt * K).astype(jnp.float32), (16,))
        res_vec = jnp.where(lane == h, num_v / den_v, res_vec)

    outv_v[...] = res_vec
    pltpu.sync_copy(outv_v, out_hbm.at[wid])


@functools.partial(jax.jit)
def _sc_topk_hitrate(keys):
    mesh = plsc.VectorSubcoreMesh(core_axis_name="c", subcore_axis_name="s",
                                  num_cores=NC, num_subcores=NS)
    return pl.kernel(
        _sc_body,
        out_type=jax.ShapeDtypeStruct((NW, 16), jnp.float32),
        mesh=mesh,
        compiler_params=pltpu.CompilerParams(needs_layout_passes=False),
        scratch_types=[
            pltpu.VMEM((HPW, N), jnp.uint32),
            pltpu.VMEM((256 * 16,), jnp.int32),
            pltpu.VMEM((257 * 16,), jnp.int32),
            pltpu.VMEM((16,), jnp.float32),
            pltpu.SemaphoreType.DMA,
            pltpu.SemaphoreType.DMA,
            pltpu.SemaphoreType.DMA,
            pltpu.SemaphoreType.DMA,
        ],
    )(keys)


def kernel(preds, targets):
    keys = _tc_transform(preds, targets)
    out = _sc_topk_hitrate(keys)
    return out[:, :HPW].reshape(H)


# speedup vs baseline: 1.4450x; 1.0996x over previous
import functools

import jax
import jax.numpy as jnp
from jax import lax
from jax.experimental import pallas as pl
from jax.experimental.pallas import tpu as pltpu
from jax.experimental.pallas import tpu_sc as plsc

N = 16384
H = 128
K = int(N * 0.1)

NC = 2
NS = 16
NSTRIPE = 8
MPS = 4
RPT = N // MPS
TCH = 4
CH = RPT // TCH
NBUF = 3


def _suffix_and_search(hist_v, s_v, lane, rank_vec):
    zero16 = jnp.zeros((16,), jnp.int32)

    def sfx(i, acc):
        br = 255 - i
        acc = acc + hist_v[br, :]
        s_v[br, :] = acc
        hist_v[br, :] = zero16
        return acc

    lax.fori_loop(0, 256, sfx, zero16, unroll=8)

    lo = jnp.zeros((16,), jnp.int32)
    hi = jnp.full((16,), 255, dtype=jnp.int32)
    for _ in range(8):
        mid = (lo + hi + 1) >> 1
        v = plsc.load_gather(s_v, [mid, lane])
        ge = (v >> 16) >= rank_vec
        lo = jnp.where(ge, mid, lo)
        hi = jnp.where(ge, hi, mid - 1)
    p = lo
    t_in = plsc.load_gather(s_v, [p, lane])
    t_ab = plsc.load_gather(s_v, [p + 1, lane])
    above = t_ab >> 16
    hits_hi = t_ab & 0xFFFF
    e_cnt = (t_in >> 16) - above
    e_pos = (t_in & 0xFFFF) - hits_hi
    return p, above, hits_hi, e_cnt, e_pos


def _sc_body(p_hbm, t_hbm, out_hbm,
             keys_v, tbuf_v, hist_v, s_v, outv_v, idx_v, shared_v,
             semp, semt):
    c = lax.axis_index("c")
    s = lax.axis_index("s")
    sid = c * 4 + s // 4
    m = s % 4
    row0 = m * RPT
    col0 = sid * 16

    cp_p = pltpu.async_copy(
        p_hbm.at[pl.ds(row0, RPT), pl.ds(col0, 16)], keys_v, semp)
    cp_t = [pltpu.async_copy(
        t_hbm.at[pl.ds(row0 + i * CH, CH), pl.ds(col0, 16)],
        tbuf_v.at[i], semt) for i in range(NBUF)]

    lane = lax.broadcasted_iota(jnp.int32, (16,), 0)
    zero16 = jnp.zeros((16,), jnp.int32)

    @plsc.parallel_loop(0, 256, 1, unroll=8)
    def _(b):
        hist_v[b, :] = zero16

    s_v[256, :] = zero16

    @plsc.parallel_loop(0, 256, 16, unroll=8)
    def _(b):
        idx_v[pl.ds(b, 16)] = sid * 256 + b + lane

    @pl.when(m == 0)
    def _():
        pltpu.sync_copy(hist_v, shared_v.at[pl.ds(sid * 256, 256)])

    cp_p.wait()
    for i in range(TCH):
        cp_t[i].wait()

        @plsc.parallel_loop(0, CH, 1, unroll=4)
        def _(r):
            row = i * CH + r
            bi = plsc.bitcast(keys_v[row, :], jnp.int32)
            mono = plsc.bitcast(bi, jnp.uint32) ^ (
                plsc.bitcast(bi >> 31, jnp.uint32) | jnp.uint32(0x80000000))
            pos_m = tbuf_v[i % NBUF, r, :] > 0.0
            w = (mono & jnp.uint32(0xFFFFFFFE)) | pos_m.astype(jnp.uint32)
            keys_v[row, :] = plsc.bitcast(w, jnp.float32)
            b1 = plsc.bitcast(w >> jnp.uint32(24), jnp.int32)
            val = jnp.where(pos_m, 0x10001, 0x10000)
            plsc.addupdate_scatter(hist_v, [b1, lane], val)

        if i + NBUF < TCH:
            cp_t.append(pltpu.async_copy(
                t_hbm.at[pl.ds(row0 + (i + NBUF) * CH, CH), pl.ds(col0, 16)],
                tbuf_v.at[(i + NBUF) % NBUF], semt))

    plsc.subcore_barrier()
    pltpu.sync_copy(hist_v, shared_v.at[idx_v], add=True)
    plsc.subcore_barrier()
    pltpu.sync_copy(shared_v.at[pl.ds(sid * 256, 256)], hist_v)

    rank0 = jnp.full((16,), K, dtype=jnp.int32)
    p1b, above1, hits1, _, _ = _suffix_and_search(hist_v, s_v, lane, rank0)
    rank1 = rank0 - above1

    @pl.when(m == 0)
    def _():
        pltpu.sync_copy(hist_v, shared_v.at[pl.ds(sid * 256, 256)])

    p1u = plsc.bitcast(p1b, jnp.uint32)

    @plsc.parallel_loop(0, RPT, 1, unroll=4)
    def _(r):
        w = plsc.bitcast(keys_v[r, :], jnp.uint32)
        b2 = plsc.bitcast((w >> jnp.uint32(16)) & jnp.uint32(0xFF),
                          jnp.int32)
        val = plsc.bitcast((w & jnp.uint32(1)) | jnp.uint32(0x10000),
                           jnp.int32)
        plsc.addupdate_scatter(hist_v, [b2, lane], val,
                               mask=(w >> jnp.uint32(24)) == p1u)

    plsc.subcore_barrier()
    pltpu.sync_copy(hist_v, shared_v.at[idx_v], add=True)
    plsc.subcore_barrier()
    pltpu.sync_copy(shared_v.at[pl.ds(sid * 256, 256)], hist_v)

    _, above2, hits2, e_cnt, e_pos = _suffix_and_search(
        hist_v, s_v, lane, rank1)
    rank2 = rank1 - above2

    num = ((hits1 + hits2) * e_cnt + rank2 * e_pos).astype(jnp.float32)
    den = (e_cnt * K).astype(jnp.float32)
    outv_v[...] = num / den

    @pl.when(m == 0)
    def _():
        pltpu.sync_copy(outv_v, out_hbm.at[sid])


@functools.partial(jax.jit)
def _sc_topk_hitrate(preds, targets):
    mesh = plsc.VectorSubcoreMesh(core_axis_name="c", subcore_axis_name="s",
                                  num_cores=NC, num_subcores=NS)
    return pl.kernel(
        _sc_body,
        out_type=jax.ShapeDtypeStruct((NSTRIPE, 16), jnp.float32),
        mesh=mesh,
        compiler_params=pltpu.CompilerParams(needs_layout_passes=False,
                                             use_tc_tiling_on_sc=False),
        scratch_types=[
            pltpu.VMEM((RPT, 16), jnp.float32),
            pltpu.VMEM((NBUF, CH, 16), jnp.float32),
            pltpu.VMEM((256, 16), jnp.int32),
            pltpu.VMEM((257, 16), jnp.int32),
            pltpu.VMEM((16,), jnp.float32),
            pltpu.VMEM((256,), jnp.int32),
            pltpu.VMEM_SHARED((NSTRIPE * 256, 16), jnp.int32),
            pltpu.SemaphoreType.DMA,
            pltpu.SemaphoreType.DMA,
        ],
    )(preds, targets)


def kernel(preds, targets):
    return _sc_topk_hitrate(preds, targets).reshape(H)


# speedup vs baseline: 1.5259x; 1.0560x over previous
import functools

import jax
import jax.numpy as jnp
from jax import lax
from jax.experimental import pallas as pl
from jax.experimental.pallas import tpu as pltpu
from jax.experimental.pallas import tpu_sc as plsc

N = 16384
H = 128
K = int(N * 0.1)

NC = 2
NS = 16
NSTRIPE = 8
MPS = 4
RPT = N // MPS
TCH = 4
CH = RPT // TCH
NBUF = 3


def _suffix_and_search(hist_v, s_v, lane, rank_vec):
    zero16 = jnp.zeros((16,), jnp.int32)

    def sfx(i, acc):
        br = 255 - i
        acc = acc + hist_v[br, :]
        s_v[br, :] = acc
        hist_v[br, :] = zero16
        return acc

    lax.fori_loop(0, 256, sfx, zero16, unroll=8)

    lo = jnp.zeros((16,), jnp.int32)
    hi = jnp.full((16,), 255, dtype=jnp.int32)
    for _ in range(8):
        mid = (lo + hi + 1) >> 1
        v = plsc.load_gather(s_v, [mid, lane])
        ge = (v >> 16) >= rank_vec
        lo = jnp.where(ge, mid, lo)
        hi = jnp.where(ge, hi, mid - 1)
    p = lo
    t_in = plsc.load_gather(s_v, [p, lane])
    t_ab = plsc.load_gather(s_v, [p + 1, lane])
    above = t_ab >> 16
    hits_hi = t_ab & 0xFFFF
    e_cnt = (t_in >> 16) - above
    e_pos = (t_in & 0xFFFF) - hits_hi
    return p, above, hits_hi, e_cnt, e_pos


def _sc_body(p_hbm, t_hbm, out_hbm,
             keys_v, tbuf_v, hist_v, s_v, outv_v, idx_v, shared_v,
             semp, semt):
    c = lax.axis_index("c")
    s = lax.axis_index("s")
    sid = c * 4 + s // 4
    m = s % 4
    row0 = m * RPT
    col0 = sid * 16

    cp_p = [pltpu.async_copy(
        p_hbm.at[pl.ds(row0 + i * CH, CH), pl.ds(col0, 16)],
        keys_v.at[pl.ds(i * CH, CH)], semp) for i in range(TCH)]
    cp_t = [pltpu.async_copy(
        t_hbm.at[pl.ds(row0 + i * CH, CH), pl.ds(col0, 16)],
        tbuf_v.at[i], semt) for i in range(NBUF)]

    lane = lax.broadcasted_iota(jnp.int32, (16,), 0)
    zero16 = jnp.zeros((16,), jnp.int32)

    @plsc.parallel_loop(0, 256, 1, unroll=8)
    def _(b):
        hist_v[b, :] = zero16

    s_v[256, :] = zero16

    @plsc.parallel_loop(0, 256, 16, unroll=8)
    def _(b):
        idx_v[pl.ds(b, 16)] = sid * 256 + b + lane

    @pl.when(m == 0)
    def _():
        pltpu.sync_copy(hist_v, shared_v.at[pl.ds(sid * 256, 256)])

    for i in range(TCH):
        cp_p[i].wait()
        cp_t[i].wait()

        @plsc.parallel_loop(0, CH, 1, unroll=4)
        def _(r):
            row = i * CH + r
            bi = plsc.bitcast(keys_v[row, :], jnp.int32)
            mono = plsc.bitcast(bi, jnp.uint32) ^ (
                plsc.bitcast(bi >> 31, jnp.uint32) | jnp.uint32(0x80000000))
            pos_m = tbuf_v[i % NBUF, r, :] > 0.0
            w = (mono & jnp.uint32(0xFFFFFFFE)) | pos_m.astype(jnp.uint32)
            keys_v[row, :] = plsc.bitcast(w, jnp.float32)
            b1 = plsc.bitcast(w >> jnp.uint32(24), jnp.int32)
            val = jnp.where(pos_m, 0x10001, 0x10000)
            plsc.addupdate_scatter(hist_v, [b1, lane], val)

        if i + NBUF < TCH:
            cp_t.append(pltpu.async_copy(
                t_hbm.at[pl.ds(row0 + (i + NBUF) * CH, CH), pl.ds(col0, 16)],
                tbuf_v.at[(i + NBUF) % NBUF], semt))

    plsc.subcore_barrier()
    pltpu.sync_copy(hist_v, shared_v.at[idx_v], add=True)
    plsc.subcore_barrier()
    pltpu.sync_copy(shared_v.at[pl.ds(sid * 256, 256)], hist_v)

    rank0 = jnp.full((16,), K, dtype=jnp.int32)
    p1b, above1, hits1, _, _ = _suffix_and_search(hist_v, s_v, lane, rank0)
    rank1 = rank0 - above1

    @pl.when(m == 0)
    def _():
        pltpu.sync_copy(hist_v, shared_v.at[pl.ds(sid * 256, 256)])

    p1u = plsc.bitcast(p1b, jnp.uint32)

    @plsc.parallel_loop(0, RPT, 1, unroll=4)
    def _(r):
        w = plsc.bitcast(keys_v[r, :], jnp.uint32)
        b2 = plsc.bitcast((w >> jnp.uint32(16)) & jnp.uint32(0xFF),
                          jnp.int32)
        val = plsc.bitcast((w & jnp.uint32(1)) | jnp.uint32(0x10000),
                           jnp.int32)
        plsc.addupdate_scatter(hist_v, [b2, lane], val,
                               mask=(w >> jnp.uint32(24)) == p1u)

    plsc.subcore_barrier()
    pltpu.sync_copy(hist_v, shared_v.at[idx_v], add=True)
    plsc.subcore_barrier()
    pltpu.sync_copy(shared_v.at[pl.ds(sid * 256, 256)], hist_v)

    _, above2, hits2, e_cnt, e_pos = _suffix_and_search(
        hist_v, s_v, lane, rank1)
    rank2 = rank1 - above2

    num = ((hits1 + hits2) * e_cnt + rank2 * e_pos).astype(jnp.float32)
    den = (e_cnt * K).astype(jnp.float32)
    outv_v[...] = num / den

    @pl.when(m == 0)
    def _():
        pltpu.sync_copy(outv_v, out_hbm.at[sid])


@functools.partial(jax.jit)
def _sc_topk_hitrate(preds, targets):
    mesh = plsc.VectorSubcoreMesh(core_axis_name="c", subcore_axis_name="s",
                                  num_cores=NC, num_subcores=NS)
    return pl.kernel(
        _sc_body,
        out_type=jax.ShapeDtypeStruct((NSTRIPE, 16), jnp.float32),
        mesh=mesh,
        compiler_params=pltpu.CompilerParams(needs_layout_passes=False,
                                             use_tc_tiling_on_sc=False),
        scratch_types=[
            pltpu.VMEM((RPT, 16), jnp.float32),
            pltpu.VMEM((NBUF, CH, 16), jnp.float32),
            pltpu.VMEM((256, 16), jnp.int32),
            pltpu.VMEM((257, 16), jnp.int32),
            pltpu.VMEM((16,), jnp.float32),
            pltpu.VMEM((256,), jnp.int32),
            pltpu.VMEM_SHARED((NSTRIPE * 256, 16), jnp.int32),
            pltpu.SemaphoreType.DMA,
            pltpu.SemaphoreType.DMA,
        ],
    )(preds, targets)


def kernel(preds, targets):
    return _sc_topk_hitrate(preds, targets).reshape(H)
